# Initial kernel scaffold; baseline (speedup 1.0000x reference)
import jax
import jax.numpy as jnp
from jax.experimental import pallas as pl


def kernel(morph0_features, W1, b1, W2, b2, W3, b3):
    raise NotImplementedError("write your pallas kernel here")


# speedup vs baseline: 1.0133x; 1.0133x over previous
import functools

import jax
import jax.numpy as jnp
from jax.experimental import pallas as pl
from jax.experimental.pallas import tpu as pltpu

F = 1024
H = 4096
BM = 512
HC = 1024
NJ = H // HC


def _body(x_ref, w1_ref, b1_ref, w2_ref, b2_ref, w3_ref, b3_ref, o_ref,
          h1_ref, acc_ref):
    j = pl.program_id(1)

    @pl.when(j == 0)
    def _():
        xb = (2.0 * x_ref[...]).astype(jnp.bfloat16)
        h1 = jnp.dot(xb, w1_ref[...], preferred_element_type=jnp.float32)
        h1 = jnp.maximum(h1 + b1_ref[...], 0.0)
        h1_ref[...] = h1.astype(jnp.bfloat16)

    h2 = jnp.dot(h1_ref[...], w2_ref[...], preferred_element_type=jnp.float32)
    h2 = jnp.maximum(h2 + b2_ref[...], 0.0).astype(jnp.bfloat16)
    contrib = jnp.dot(h2, w3_ref[...], preferred_element_type=jnp.float32)

    @pl.when(j == 0)
    def _():
        acc_ref[...] = contrib

    @pl.when(j > 0)
    def _():
        acc_ref[...] += contrib

    @pl.when(j == NJ - 1)
    def _():
        o_ref[...] = x_ref[...] + acc_ref[...] + b3_ref[...]


@functools.partial(jax.jit, static_argnums=())
def kernel(morph0_features, W1, b1, W2, b2, W3, b3):
    B = morph0_features.shape[0]
    w1b = W1.astype(jnp.bfloat16)
    w2b = W2.astype(jnp.bfloat16)
    w3b = W3.astype(jnp.bfloat16)
    b1r = b1.reshape(1, H)
    b2r = b2.reshape(1, H)
    b3r = b3.reshape(1, F)

    grid = (B // BM, NJ)
    out = pl.pallas_call(
        _body,
        grid=grid,
        in_specs=[
            pl.BlockSpec((BM, F), lambda i, j: (i, 0)),
            pl.BlockSpec((F, H), lambda i, j: (0, 0)),
            pl.BlockSpec((1, H), lambda i, j: (0, 0)),
            pl.BlockSpec((H, HC), lambda i, j: (0, j)),
            pl.BlockSpec((1, HC), lambda i, j: (0, j)),
            pl.BlockSpec((HC, F), lambda i, j: (j, 0)),
            pl.BlockSpec((1, F), lambda i, j: (0, 0)),
        ],
        out_specs=pl.BlockSpec((BM, F), lambda i, j: (i, 0)),
        out_shape=jax.ShapeDtypeStruct((B, F), jnp.float32),
        scratch_shapes=[
            pltpu.VMEM((BM, H), jnp.bfloat16),
            pltpu.VMEM((BM, F), jnp.float32),
        ],
        compiler_params=pltpu.CompilerParams(
            dimension_semantics=("parallel", "arbitrary"),
        ),
    )(morph0_features, w1b, b1r, w2b, b2r, w3b, b3r)
    return out


# speedup vs baseline: 1.0400x; 1.0264x over previous
import functools

import jax
import jax.numpy as jnp
from jax.experimental import pallas as pl
from jax.experimental.pallas import tpu as pltpu

F = 1024
H = 4096
BM = 512
HC = 1024
NJ = H // HC


def _body(x_ref, w1_ref, b1_ref, w2_ref, b2_ref, w3_ref, b3_ref, o_ref,
          h1_ref):
    j = pl.program_id(1)

    @pl.when(j == 0)
    def _():
        xb = (2.0 * x_ref[...]).astype(jnp.bfloat16)
        h1 = jnp.dot(xb, w1_ref[...], preferred_element_type=jnp.float32)
        h1 = jnp.maximum(h1 + b1_ref[...], 0.0)
        h1_ref[...] = h1.astype(jnp.bfloat16)
        o_ref[...] = x_ref[...] + b3_ref[...]

    h2 = jnp.dot(h1_ref[...], w2_ref[...], preferred_element_type=jnp.float32)
    h2 = jnp.maximum(h2 + b2_ref[...], 0.0).astype(jnp.bfloat16)
    o_ref[...] += jnp.dot(h2, w3_ref[...], preferred_element_type=jnp.float32)


@functools.partial(jax.jit, static_argnums=())
def kernel(morph0_features, W1, b1, W2, b2, W3, b3):
    B = morph0_features.shape[0]
    w1b = W1.astype(jnp.bfloat16)
    w2b = W2.astype(jnp.bfloat16)
    w3b = W3.astype(jnp.bfloat16)
    b1r = b1.reshape(1, H)
    b2r = b2.reshape(1, H)
    b3r = b3.reshape(1, F)

    grid = (B // BM, NJ)
    out = pl.pallas_call(
        _body,
        grid=grid,
        in_specs=[
            pl.BlockSpec((BM, F), lambda i, j: (i, 0)),
            pl.BlockSpec((F, H), lambda i, j: (0, 0)),
            pl.BlockSpec((1, H), lambda i, j: (0, 0)),
            pl.BlockSpec((H, HC), lambda i, j: (0, j)),
            pl.BlockSpec((1, HC), lambda i, j: (0, j)),
            pl.BlockSpec((HC, F), lambda i, j: (j, 0)),
            pl.BlockSpec((1, F), lambda i, j: (0, 0)),
        ],
        out_specs=pl.BlockSpec((BM, F), lambda i, j: (i, 0)),
        out_shape=jax.ShapeDtypeStruct((B, F), jnp.float32),
        scratch_shapes=[
            pltpu.VMEM((BM, H), jnp.bfloat16),
        ],
        compiler_params=pltpu.CompilerParams(
            dimension_semantics=("parallel", "arbitrary"),
        ),
    )(morph0_features, w1b, b1r, w2b, b2r, w3b, b3r)
    return out
